# baseline (speedup 1.0000x reference)
import functools

import jax
import jax.numpy as jnp
from jax import lax
from jax.experimental import pallas as pl
from jax.experimental.pallas import tpu as pltpu
from jax.experimental.pallas import tpu_sc as plsc

NC = 2
NS = 16
NW = NC * NS
G = 64
ECH = 128


def _round_up(a, m):
    return (a + m - 1) // m * m


FAST = 0
EMB_CH = 80
EMB_NF = 6
EMB_NS = 2


def _emb_gather(emb, idx, n_pad, d):
    mesh = plsc.VectorSubcoreMesh(core_axis_name="c", subcore_axis_name="s",
                                  num_cores=NC, num_subcores=NS)

    @functools.partial(
        pl.kernel,
        out_type=jax.ShapeDtypeStruct((n_pad, d), jnp.float32),
        mesh=mesh,
        scratch_types=[
            pltpu.VMEM((EMB_CH,), jnp.int32),
            pltpu.VMEM((EMB_CH, d), jnp.float32),
        ],
    )
    def k(emb_hbm, idx_hbm, h_hbm, idxv, rows):
        cid = lax.axis_index("c")
        sid = lax.axis_index("s")

        def run(base, nch):
            for c in range(nch):
                off = base + c * EMB_CH
                pltpu.sync_copy(idx_hbm.at[pl.ds(off, EMB_CH)], idxv)
                pltpu.sync_copy(emb_hbm.at[idxv], rows)
                pltpu.sync_copy(rows, h_hbm.at[pl.ds(off, EMB_CH)])

        @pl.when(cid == FAST)
        def _():
            run(sid * (EMB_NF * EMB_CH), EMB_NF)

        @pl.when(cid != FAST)
        def _():
            run(NS * EMB_NF * EMB_CH + sid * (EMB_NS * EMB_CH), EMB_NS)

    return k(emb, idx)


def _edge_agg(h_pad, edges_packed, n_pad, d, stage):
    nch_total = edges_packed.shape[0]
    nchw = nch_total // NS
    rps = n_pad // NS
    mesh = plsc.VectorSubcoreMesh(core_axis_name="c", subcore_axis_name="s",
                                  num_cores=NC, num_subcores=NS)

    qmax = 40

    @functools.partial(
        pl.kernel,
        out_type=(jax.ShapeDtypeStruct((n_pad, d), jnp.float32),
                  jax.ShapeDtypeStruct((n_pad, d), jnp.float32)),
        mesh=mesh,
        scratch_types=[
            pltpu.VMEM((qmax, 2, ECH), jnp.int32),
            pltpu.VMEM((ECH, d), jnp.float32),
            pltpu.VMEM((ECH, d), jnp.float32),
            pltpu.VMEM_SHARED((n_pad, d), jnp.float32),
            pltpu.SemaphoreType.DMA,
            pltpu.SemaphoreType.DMA,
        ],
    )
    def k(h_in_hbm, ei_hbm, out_hbm, hl_hbm, iall, ra, rb, aggsh, sa, sb):
        cid = lax.axis_index("c")
        sid = lax.axis_index("s")
        h_hbm = hl_hbm if stage else h_in_hbm

        def pipe_q(qoff, qn):
            pltpu.sync_copy(ei_hbm.at[pl.ds(qoff, qn)],
                            iall.at[pl.ds(0, qn)])
            pltpu.async_copy(h_hbm.at[iall.at[0, 0]], ra, sa)

            @pl.loop(0, (qn - 2) // 2)
            def _(j):
                a = 2 * j
                pltpu.async_copy(h_hbm.at[iall.at[a + 1, 0]], rb, sb)
                pltpu.make_async_copy(h_hbm.at[iall.at[a, 0]], ra, sa).wait()
                pltpu.sync_copy(ra, aggsh.at[iall.at[a, 1]], add=True)
                pltpu.async_copy(h_hbm.at[iall.at[a + 2, 0]], ra, sa)
                pltpu.make_async_copy(h_hbm.at[iall.at[a + 1, 0]], rb, sb).wait()
                pltpu.sync_copy(rb, aggsh.at[iall.at[a + 1, 1]], add=True)

            pltpu.async_copy(h_hbm.at[iall.at[qn - 1, 0]], rb, sb)
            pltpu.make_async_copy(h_hbm.at[iall.at[qn - 2, 0]], ra, sa).wait()
            pltpu.sync_copy(ra, aggsh.at[iall.at[qn - 2, 1]], add=True)
            pltpu.make_async_copy(h_hbm.at[iall.at[qn - 1, 0]], rb, sb).wait()
            pltpu.sync_copy(rb, aggsh.at[iall.at[qn - 1, 1]], add=True)

        def pipe(base, nchw):
            done = 0
            while done < nchw:
                qn = min(qmax, nchw - done)
                pipe_q(base + done, qn)
                done += qn

        @pl.when(cid == FAST)
        def _():
            if stage:
                pltpu.sync_copy(h_in_hbm.at[pl.ds(sid * rps, rps)],
                                hl_hbm.at[pl.ds(sid * rps, rps)])
            zv = jnp.zeros((16,), jnp.float32)

            @pl.loop(0, ECH)
            def _(i):
                for j in range(d // 16):
                    ra[i, pl.ds(j * 16, 16)] = zv

            for r in range(rps // ECH):
                pltpu.sync_copy(ra, aggsh.at[pl.ds(sid * rps + r * ECH, ECH)])
            plsc.subcore_barrier()
            pipe(sid * nchw, nchw)
            plsc.subcore_barrier()
            pltpu.sync_copy(aggsh.at[pl.ds(sid * rps, rps)],
                            out_hbm.at[pl.ds(sid * rps, rps)])

    return k(h_pad, edges_packed)[0]


def _mlp_body(h_ref, a_ref, w1_ref, b1_ref, w2_ref, b2_ref, sc_ref, o_ref):
    s = sc_ref[0, 0]
    z = h_ref[...] * s + a_ref[...]
    z = jnp.maximum(
        jnp.dot(z, w1_ref[...], preferred_element_type=jnp.float32)
        + b1_ref[...], 0.0)
    z = jnp.maximum(
        jnp.dot(z, w2_ref[...], preferred_element_type=jnp.float32)
        + b2_ref[...], 0.0)
    o_ref[...] = z


def _mlp(h_pad, agg2, w1, b1, w2, b2, scale, n_pad, d, blk):
    nb = n_pad // blk
    return pl.pallas_call(
        _mlp_body,
        grid=(nb,),
        in_specs=[
            pl.BlockSpec((blk, d), lambda i: (i, 0)),
            pl.BlockSpec((blk, d), lambda i: (i, 0)),
            pl.BlockSpec((d, d), lambda i: (0, 0)),
            pl.BlockSpec((1, d), lambda i: (0, 0)),
            pl.BlockSpec((d, d), lambda i: (0, 0)),
            pl.BlockSpec((1, d), lambda i: (0, 0)),
            pl.BlockSpec(memory_space=pltpu.SMEM),
        ],
        out_specs=pl.BlockSpec((blk, d), lambda i: (i, 0)),
        out_shape=jax.ShapeDtypeStruct((n_pad, d), jnp.float32),
    )(h_pad, agg2, w1, b1, w2, b2, scale)


def _mlp_pool_cls(h_pad, agg2, w1, b1, w2, b2, scale, batch3, wc1, bc1,
                  wc2, bc2, n_pad, d, blk):
    nb = n_pad // blk

    def body(h_ref, a_ref, w1_ref, b1_ref, w2_ref, b2_ref, sc_ref, bt_ref,
             wc1_ref, bc1_ref, wc2_ref, bc2_ref, s_ref, hg_ref):
        i = pl.program_id(0)
        s = sc_ref[0, 0]
        z = h_ref[...] * s + a_ref[...]
        z = jnp.maximum(
            jnp.dot(z, w1_ref[...], preferred_element_type=jnp.float32)
            + b1_ref[...], 0.0)
        z = jnp.maximum(
            jnp.dot(z, w2_ref[...], preferred_element_type=jnp.float32)
            + b2_ref[...], 0.0)
        bt = bt_ref[0, 0, :]
        oh = (bt[:, None] == lax.broadcasted_iota(jnp.int32, (blk, G), 1))
        oh = oh.astype(jnp.float32)
        contrib = lax.dot_general(
            oh, z, (((0,), (0,)), ((), ())),
            preferred_element_type=jnp.float32)

        @pl.when(i == 0)
        def _():
            hg_ref[...] = jnp.zeros_like(hg_ref)

        hg_ref[...] += contrib

        @pl.when(i == nb - 1)
        def _():
            hg = hg_ref[...]
            hid = jnp.maximum(
                jnp.dot(hg, wc1_ref[...], preferred_element_type=jnp.float32)
                + bc1_ref[...], 0.0)
            logit = jnp.dot(hid, wc2_ref[...],
                            preferred_element_type=jnp.float32) + bc2_ref[0, 0]
            s_ref[...] = jax.nn.sigmoid(logit)

    return pl.pallas_call(
        body,
        grid=(nb,),
        in_specs=[
            pl.BlockSpec((blk, d), lambda i: (i, 0)),
            pl.BlockSpec((blk, d), lambda i: (i, 0)),
            pl.BlockSpec((d, d), lambda i: (0, 0)),
            pl.BlockSpec((1, d), lambda i: (0, 0)),
            pl.BlockSpec((d, d), lambda i: (0, 0)),
            pl.BlockSpec((1, d), lambda i: (0, 0)),
            pl.BlockSpec(memory_space=pltpu.SMEM),
            pl.BlockSpec((1, 1, blk), lambda i: (i, 0, 0)),
            pl.BlockSpec((d, d), lambda i: (0, 0)),
            pl.BlockSpec((1, d), lambda i: (0, 0)),
            pl.BlockSpec((d, 1), lambda i: (0, 0)),
            pl.BlockSpec(memory_space=pltpu.SMEM),
        ],
        out_specs=pl.BlockSpec((G, 1), lambda i: (0, 0)),
        out_shape=jax.ShapeDtypeStruct((G, 1), jnp.float32),
        scratch_shapes=[pltpu.VMEM((G, d), jnp.float32)],
    )(h_pad, agg2, w1, b1, w2, b2, scale, batch3, wc1, bc1, wc2, bc2)


def kernel(x, edge_index, batch, emb, W1, b1, W2, b2, eps, Wc1, bc1, Wc2, bc2):
    n = x.shape[0]
    d = emb.shape[1]
    e = edge_index.shape[1]
    n_layers = W1.shape[0]
    blk = 1024

    n_pad = _round_up(n + 1, NW * 80)
    e_pad = _round_up(e, NS * ECH * 2)

    idx = jnp.concatenate(
        [x[:, 0], jnp.zeros((n_pad - n,), jnp.int32)])
    ei_pad = jnp.concatenate(
        [edge_index, jnp.full((2, e_pad - e), n, jnp.int32)], axis=1)
    edges_packed = ei_pad.reshape(2, e_pad // ECH, ECH).transpose(1, 0, 2)
    batch3 = jnp.concatenate(
        [batch, jnp.full((n_pad - n,), G, jnp.int32)]).reshape(
            n_pad // blk, 1, blk)

    h = _emb_gather(emb, idx, n_pad, d)
    for l in range(n_layers):
        agg2 = _edge_agg(h, edges_packed, n_pad, d, stage=(l > 0))
        scale = (1.0 + eps[l]).reshape(1, 1)
        b1l = b1[l].reshape(1, d)
        b2l = b2[l].reshape(1, d)
        if l < n_layers - 1:
            h = _mlp(h, agg2, W1[l], b1l, W2[l], b2l, scale, n_pad, d, blk)
        else:
            score = _mlp_pool_cls(
                h, agg2, W1[l], b1l, W2[l], b2l, scale, batch3,
                Wc1, bc1.reshape(1, d), Wc2, bc2.reshape(1, 1),
                n_pad, d, blk)
    return score.reshape(-1)


# speedup vs baseline: 1.2030x; 1.2030x over previous
import functools

import jax
import jax.numpy as jnp
from jax import lax
from jax.experimental import pallas as pl
from jax.experimental.pallas import tpu as pltpu
from jax.experimental.pallas import tpu_sc as plsc

NC = 2
NS = 16
NW = NC * NS
G = 64
ECH = 128


def _round_up(a, m):
    return (a + m - 1) // m * m


FAST = 0
EMB_CH = 80
EMB_NF = 6
EMB_NS = 2


def _emb_gather(emb, idx, n_pad, d):
    mesh = plsc.VectorSubcoreMesh(core_axis_name="c", subcore_axis_name="s",
                                  num_cores=NC, num_subcores=NS)

    @functools.partial(
        pl.kernel,
        out_type=jax.ShapeDtypeStruct((n_pad, d), jnp.float32),
        mesh=mesh,
        scratch_types=[
            pltpu.VMEM((EMB_CH,), jnp.int32),
            pltpu.VMEM((EMB_CH, d), jnp.float32),
        ],
    )
    def k(emb_hbm, idx_hbm, h_hbm, idxv, rows):
        cid = lax.axis_index("c")
        sid = lax.axis_index("s")

        def run(base, nch):
            for c in range(nch):
                off = base + c * EMB_CH
                pltpu.sync_copy(idx_hbm.at[pl.ds(off, EMB_CH)], idxv)
                pltpu.sync_copy(emb_hbm.at[idxv], rows)
                pltpu.sync_copy(rows, h_hbm.at[pl.ds(off, EMB_CH)])

        @pl.when(cid == FAST)
        def _():
            run(sid * (EMB_NF * EMB_CH), EMB_NF)

        @pl.when(cid != FAST)
        def _():
            run(NS * EMB_NF * EMB_CH + sid * (EMB_NS * EMB_CH), EMB_NS)

    return k(emb, idx)


def _edge_agg(h_pad, edges_packed, n_pad, d):
    nch_total = edges_packed.shape[0]
    nchw = nch_total // NS
    rps = n_pad // NS
    mesh = plsc.VectorSubcoreMesh(core_axis_name="c", subcore_axis_name="s",
                                  num_cores=NC, num_subcores=NS)

    qmax = 40

    @functools.partial(
        pl.kernel,
        out_type=jax.ShapeDtypeStruct((n_pad, d), jnp.float32),
        mesh=mesh,
        scratch_types=[
            pltpu.VMEM((qmax, 2, ECH), jnp.int32),
            pltpu.VMEM((ECH, d), jnp.float32),
            pltpu.VMEM((ECH, d), jnp.float32),
            pltpu.VMEM_SHARED((n_pad, d), jnp.float32),
            pltpu.SemaphoreType.DMA,
            pltpu.SemaphoreType.DMA,
        ],
    )
    def k(h_hbm, ei_hbm, out_hbm, iall, ra, rb, aggsh, sa, sb):
        cid = lax.axis_index("c")
        sid = lax.axis_index("s")

        def pipe_q(qoff, qn):
            pltpu.sync_copy(ei_hbm.at[pl.ds(qoff, qn)],
                            iall.at[pl.ds(0, qn)])
            pltpu.async_copy(h_hbm.at[iall.at[0, 0]], ra, sa)

            @pl.loop(0, (qn - 2) // 2)
            def _(j):
                a = 2 * j
                pltpu.async_copy(h_hbm.at[iall.at[a + 1, 0]], rb, sb)
                pltpu.make_async_copy(h_hbm.at[iall.at[a, 0]], ra, sa).wait()
                pltpu.sync_copy(ra, aggsh.at[iall.at[a, 1]], add=True)
                pltpu.async_copy(h_hbm.at[iall.at[a + 2, 0]], ra, sa)
                pltpu.make_async_copy(h_hbm.at[iall.at[a + 1, 0]], rb, sb).wait()
                pltpu.sync_copy(rb, aggsh.at[iall.at[a + 1, 1]], add=True)

            pltpu.async_copy(h_hbm.at[iall.at[qn - 1, 0]], rb, sb)
            pltpu.make_async_copy(h_hbm.at[iall.at[qn - 2, 0]], ra, sa).wait()
            pltpu.sync_copy(ra, aggsh.at[iall.at[qn - 2, 1]], add=True)
            pltpu.make_async_copy(h_hbm.at[iall.at[qn - 1, 0]], rb, sb).wait()
            pltpu.sync_copy(rb, aggsh.at[iall.at[qn - 1, 1]], add=True)

        def pipe(base, nchw):
            done = 0
            while done < nchw:
                qn = min(qmax, nchw - done)
                pipe_q(base + done, qn)
                done += qn

        @pl.when(cid == FAST)
        def _():
            zv = jnp.zeros((16,), jnp.float32)

            @pl.loop(0, ECH)
            def _(i):
                for j in range(d // 16):
                    ra[i, pl.ds(j * 16, 16)] = zv

            for r in range(rps // ECH):
                pltpu.sync_copy(ra, aggsh.at[pl.ds(sid * rps + r * ECH, ECH)])
            plsc.subcore_barrier()
            pipe(sid * nchw, nchw)
            plsc.subcore_barrier()
            pltpu.sync_copy(aggsh.at[pl.ds(sid * rps, rps)],
                            out_hbm.at[pl.ds(sid * rps, rps)])

    return k(h_pad, edges_packed)


def _mlp_body(h_ref, a_ref, w1_ref, b1_ref, w2_ref, b2_ref, sc_ref, o_ref):
    s = sc_ref[0, 0]
    z = h_ref[...] * s + a_ref[...]
    z = jnp.maximum(
        jnp.dot(z, w1_ref[...], preferred_element_type=jnp.float32)
        + b1_ref[...], 0.0)
    z = jnp.maximum(
        jnp.dot(z, w2_ref[...], preferred_element_type=jnp.float32)
        + b2_ref[...], 0.0)
    o_ref[...] = z


def _mlp(h_pad, agg2, w1, b1, w2, b2, scale, n_pad, d, blk):
    nb = n_pad // blk
    return pl.pallas_call(
        _mlp_body,
        grid=(nb,),
        in_specs=[
            pl.BlockSpec((blk, d), lambda i: (i, 0)),
            pl.BlockSpec((blk, d), lambda i: (i, 0)),
            pl.BlockSpec((d, d), lambda i: (0, 0)),
            pl.BlockSpec((1, d), lambda i: (0, 0)),
            pl.BlockSpec((d, d), lambda i: (0, 0)),
            pl.BlockSpec((1, d), lambda i: (0, 0)),
            pl.BlockSpec(memory_space=pltpu.SMEM),
        ],
        out_specs=pl.BlockSpec((blk, d), lambda i: (i, 0)),
        out_shape=jax.ShapeDtypeStruct((n_pad, d), jnp.float32),
    )(h_pad, agg2, w1, b1, w2, b2, scale)


def _mlp_pool_cls(h_pad, agg2, w1, b1, w2, b2, scale, batch3, wc1, bc1,
                  wc2, bc2, n_pad, d, blk):
    nb = n_pad // blk

    def body(h_ref, a_ref, w1_ref, b1_ref, w2_ref, b2_ref, sc_ref, bt_ref,
             wc1_ref, bc1_ref, wc2_ref, bc2_ref, s_ref, hg_ref):
        i = pl.program_id(0)
        s = sc_ref[0, 0]
        z = h_ref[...] * s + a_ref[...]
        z = jnp.maximum(
            jnp.dot(z, w1_ref[...], preferred_element_type=jnp.float32)
            + b1_ref[...], 0.0)
        z = jnp.maximum(
            jnp.dot(z, w2_ref[...], preferred_element_type=jnp.float32)
            + b2_ref[...], 0.0)
        bt = bt_ref[0, 0, :]
        oh = (bt[:, None] == lax.broadcasted_iota(jnp.int32, (blk, G), 1))
        oh = oh.astype(jnp.float32)
        contrib = lax.dot_general(
            oh, z, (((0,), (0,)), ((), ())),
            preferred_element_type=jnp.float32)

        @pl.when(i == 0)
        def _():
            hg_ref[...] = jnp.zeros_like(hg_ref)

        hg_ref[...] += contrib

        @pl.when(i == nb - 1)
        def _():
            hg = hg_ref[...]
            hid = jnp.maximum(
                jnp.dot(hg, wc1_ref[...], preferred_element_type=jnp.float32)
                + bc1_ref[...], 0.0)
            logit = jnp.dot(hid, wc2_ref[...],
                            preferred_element_type=jnp.float32) + bc2_ref[0, 0]
            s_ref[...] = jax.nn.sigmoid(logit)

    return pl.pallas_call(
        body,
        grid=(nb,),
        in_specs=[
            pl.BlockSpec((blk, d), lambda i: (i, 0)),
            pl.BlockSpec((blk, d), lambda i: (i, 0)),
            pl.BlockSpec((d, d), lambda i: (0, 0)),
            pl.BlockSpec((1, d), lambda i: (0, 0)),
            pl.BlockSpec((d, d), lambda i: (0, 0)),
            pl.BlockSpec((1, d), lambda i: (0, 0)),
            pl.BlockSpec(memory_space=pltpu.SMEM),
            pl.BlockSpec((1, 1, blk), lambda i: (i, 0, 0)),
            pl.BlockSpec((d, d), lambda i: (0, 0)),
            pl.BlockSpec((1, d), lambda i: (0, 0)),
            pl.BlockSpec((d, 1), lambda i: (0, 0)),
            pl.BlockSpec(memory_space=pltpu.SMEM),
        ],
        out_specs=pl.BlockSpec((G, 1), lambda i: (0, 0)),
        out_shape=jax.ShapeDtypeStruct((G, 1), jnp.float32),
        scratch_shapes=[pltpu.VMEM((G, d), jnp.float32)],
    )(h_pad, agg2, w1, b1, w2, b2, scale, batch3, wc1, bc1, wc2, bc2)


def kernel(x, edge_index, batch, emb, W1, b1, W2, b2, eps, Wc1, bc1, Wc2, bc2):
    n = x.shape[0]
    d = emb.shape[1]
    e = edge_index.shape[1]
    n_layers = W1.shape[0]
    blk = 1024

    n_pad = _round_up(n + 1, NW * 80)
    e_pad = _round_up(e, NS * ECH * 2)

    idx = jnp.concatenate(
        [x[:, 0], jnp.zeros((n_pad - n,), jnp.int32)])
    ei_pad = jnp.concatenate(
        [edge_index, jnp.full((2, e_pad - e), n, jnp.int32)], axis=1)
    edges_packed = ei_pad.reshape(2, e_pad // ECH, ECH).transpose(1, 0, 2)
    batch3 = jnp.concatenate(
        [batch, jnp.full((n_pad - n,), G, jnp.int32)]).reshape(
            n_pad // blk, 1, blk)

    h = _emb_gather(emb, idx, n_pad, d)
    for l in range(n_layers):
        agg2 = _edge_agg(h, edges_packed, n_pad, d)
        scale = (1.0 + eps[l]).reshape(1, 1)
        b1l = b1[l].reshape(1, d)
        b2l = b2[l].reshape(1, d)
        if l < n_layers - 1:
            h = _mlp(h, agg2, W1[l], b1l, W2[l], b2l, scale, n_pad, d, blk)
        else:
            score = _mlp_pool_cls(
                h, agg2, W1[l], b1l, W2[l], b2l, scale, batch3,
                Wc1, bc1.reshape(1, d), Wc2, bc2.reshape(1, 1),
                n_pad, d, blk)
    return score.reshape(-1)


# speedup vs baseline: 1.2084x; 1.0045x over previous
import functools

import jax
import jax.numpy as jnp
from jax import lax
from jax.experimental import pallas as pl
from jax.experimental.pallas import tpu as pltpu
from jax.experimental.pallas import tpu_sc as plsc

NC = 2
NS = 16
NW = NC * NS
G = 64
ECH = 128


def _round_up(a, m):
    return (a + m - 1) // m * m


FAST = 0
EMB_CH = 80
EMB_NF = 6
EMB_NS = 2


def _emb_gather(emb, idx, n_pad, d):
    mesh = plsc.VectorSubcoreMesh(core_axis_name="c", subcore_axis_name="s",
                                  num_cores=NC, num_subcores=NS)

    @functools.partial(
        pl.kernel,
        out_type=jax.ShapeDtypeStruct((n_pad, d), jnp.float32),
        mesh=mesh,
        scratch_types=[
            pltpu.VMEM((EMB_CH,), jnp.int32),
            pltpu.VMEM((EMB_CH, d), jnp.float32),
        ],
    )
    def k(emb_hbm, idx_hbm, h_hbm, idxv, rows):
        cid = lax.axis_index("c")
        sid = lax.axis_index("s")

        def run(base, nch):
            for c in range(nch):
                off = base + c * EMB_CH
                pltpu.sync_copy(idx_hbm.at[pl.ds(off, EMB_CH)], idxv)
                pltpu.sync_copy(emb_hbm.at[idxv], rows)
                pltpu.sync_copy(rows, h_hbm.at[pl.ds(off, EMB_CH)])

        @pl.when(cid == FAST)
        def _():
            run(sid * (EMB_NF * EMB_CH), EMB_NF)

        @pl.when(cid != FAST)
        def _():
            run(NS * EMB_NF * EMB_CH + sid * (EMB_NS * EMB_CH), EMB_NS)

    return k(emb, idx)


def _edge_agg(h_pad, edges_packed, n_pad, d):
    nch_total = edges_packed.shape[0]
    nchw = nch_total // NS
    rps = n_pad // NS
    mesh = plsc.VectorSubcoreMesh(core_axis_name="c", subcore_axis_name="s",
                                  num_cores=NC, num_subcores=NS)

    qmax = 60

    @functools.partial(
        pl.kernel,
        out_type=jax.ShapeDtypeStruct((n_pad, d), jnp.float32),
        mesh=mesh,
        scratch_types=[
            pltpu.VMEM((qmax, 2, ECH), jnp.int32),
            pltpu.VMEM((ECH, d), jnp.float32),
            pltpu.VMEM((ECH, d), jnp.float32),
            pltpu.VMEM_SHARED((n_pad, d), jnp.float32),
            pltpu.SemaphoreType.DMA,
            pltpu.SemaphoreType.DMA,
        ],
    )
    def k(h_hbm, ei_hbm, out_hbm, iall, ra, rb, aggsh, sa, sb):
        cid = lax.axis_index("c")
        sid = lax.axis_index("s")

        def pipe_q(qoff, qn):
            pltpu.sync_copy(ei_hbm.at[pl.ds(qoff, qn)],
                            iall.at[pl.ds(0, qn)])
            pltpu.async_copy(h_hbm.at[iall.at[0, 0]], ra, sa)

            @pl.loop(0, (qn - 2) // 2)
            def _(j):
                a = 2 * j
                pltpu.async_copy(h_hbm.at[iall.at[a + 1, 0]], rb, sb)
                pltpu.make_async_copy(h_hbm.at[iall.at[a, 0]], ra, sa).wait()
                pltpu.sync_copy(ra, aggsh.at[iall.at[a, 1]], add=True)
                pltpu.async_copy(h_hbm.at[iall.at[a + 2, 0]], ra, sa)
                pltpu.make_async_copy(h_hbm.at[iall.at[a + 1, 0]], rb, sb).wait()
                pltpu.sync_copy(rb, aggsh.at[iall.at[a + 1, 1]], add=True)

            pltpu.async_copy(h_hbm.at[iall.at[qn - 1, 0]], rb, sb)
            pltpu.make_async_copy(h_hbm.at[iall.at[qn - 2, 0]], ra, sa).wait()
            pltpu.sync_copy(ra, aggsh.at[iall.at[qn - 2, 1]], add=True)
            pltpu.make_async_copy(h_hbm.at[iall.at[qn - 1, 0]], rb, sb).wait()
            pltpu.sync_copy(rb, aggsh.at[iall.at[qn - 1, 1]], add=True)

        def pipe(base, nchw):
            done = 0
            while done < nchw:
                qn = min(qmax, nchw - done)
                pipe_q(base + done, qn)
                done += qn

        @pl.when(cid == FAST)
        def _():
            zv = jnp.zeros((16,), jnp.float32)

            @pl.loop(0, ECH)
            def _(i):
                for j in range(d // 16):
                    ra[i, pl.ds(j * 16, 16)] = zv

            for r in range(rps // ECH):
                pltpu.sync_copy(ra, aggsh.at[pl.ds(sid * rps + r * ECH, ECH)])
            plsc.subcore_barrier()
            pipe(sid * nchw, nchw)
            plsc.subcore_barrier()
            pltpu.sync_copy(aggsh.at[pl.ds(sid * rps, rps)],
                            out_hbm.at[pl.ds(sid * rps, rps)])

    return k(h_pad, edges_packed)


def _mlp_body(h_ref, a_ref, w1_ref, b1_ref, w2_ref, b2_ref, sc_ref, o_ref):
    s = sc_ref[0, 0]
    z = h_ref[...] * s + a_ref[...]
    z = jnp.maximum(
        jnp.dot(z, w1_ref[...], preferred_element_type=jnp.float32)
        + b1_ref[...], 0.0)
    z = jnp.maximum(
        jnp.dot(z, w2_ref[...], preferred_element_type=jnp.float32)
        + b2_ref[...], 0.0)
    o_ref[...] = z


def _mlp(h_pad, agg2, w1, b1, w2, b2, scale, n_pad, d, blk):
    nb = n_pad // blk
    return pl.pallas_call(
        _mlp_body,
        grid=(nb,),
        in_specs=[
            pl.BlockSpec((blk, d), lambda i: (i, 0)),
            pl.BlockSpec((blk, d), lambda i: (i, 0)),
            pl.BlockSpec((d, d), lambda i: (0, 0)),
            pl.BlockSpec((1, d), lambda i: (0, 0)),
            pl.BlockSpec((d, d), lambda i: (0, 0)),
            pl.BlockSpec((1, d), lambda i: (0, 0)),
            pl.BlockSpec(memory_space=pltpu.SMEM),
        ],
        out_specs=pl.BlockSpec((blk, d), lambda i: (i, 0)),
        out_shape=jax.ShapeDtypeStruct((n_pad, d), jnp.float32),
    )(h_pad, agg2, w1, b1, w2, b2, scale)


def _mlp_pool_cls(h_pad, agg2, w1, b1, w2, b2, scale, batch3, wc1, bc1,
                  wc2, bc2, n_pad, d, blk):
    nb = n_pad // blk

    def body(h_ref, a_ref, w1_ref, b1_ref, w2_ref, b2_ref, sc_ref, bt_ref,
             wc1_ref, bc1_ref, wc2_ref, bc2_ref, s_ref, hg_ref):
        i = pl.program_id(0)
        s = sc_ref[0, 0]
        z = h_ref[...] * s + a_ref[...]
        z = jnp.maximum(
            jnp.dot(z, w1_ref[...], preferred_element_type=jnp.float32)
            + b1_ref[...], 0.0)
        z = jnp.maximum(
            jnp.dot(z, w2_ref[...], preferred_element_type=jnp.float32)
            + b2_ref[...], 0.0)
        bt = bt_ref[0, 0, :]
        oh = (bt[:, None] == lax.broadcasted_iota(jnp.int32, (blk, G), 1))
        oh = oh.astype(jnp.float32)
        contrib = lax.dot_general(
            oh, z, (((0,), (0,)), ((), ())),
            preferred_element_type=jnp.float32)

        @pl.when(i == 0)
        def _():
            hg_ref[...] = jnp.zeros_like(hg_ref)

        hg_ref[...] += contrib

        @pl.when(i == nb - 1)
        def _():
            hg = hg_ref[...]
            hid = jnp.maximum(
                jnp.dot(hg, wc1_ref[...], preferred_element_type=jnp.float32)
                + bc1_ref[...], 0.0)
            logit = jnp.dot(hid, wc2_ref[...],
                            preferred_element_type=jnp.float32) + bc2_ref[0, 0]
            s_ref[...] = jax.nn.sigmoid(logit)

    return pl.pallas_call(
        body,
        grid=(nb,),
        in_specs=[
            pl.BlockSpec((blk, d), lambda i: (i, 0)),
            pl.BlockSpec((blk, d), lambda i: (i, 0)),
            pl.BlockSpec((d, d), lambda i: (0, 0)),
            pl.BlockSpec((1, d), lambda i: (0, 0)),
            pl.BlockSpec((d, d), lambda i: (0, 0)),
            pl.BlockSpec((1, d), lambda i: (0, 0)),
            pl.BlockSpec(memory_space=pltpu.SMEM),
            pl.BlockSpec((1, 1, blk), lambda i: (i, 0, 0)),
            pl.BlockSpec((d, d), lambda i: (0, 0)),
            pl.BlockSpec((1, d), lambda i: (0, 0)),
            pl.BlockSpec((d, 1), lambda i: (0, 0)),
            pl.BlockSpec(memory_space=pltpu.SMEM),
        ],
        out_specs=pl.BlockSpec((G, 1), lambda i: (0, 0)),
        out_shape=jax.ShapeDtypeStruct((G, 1), jnp.float32),
        scratch_shapes=[pltpu.VMEM((G, d), jnp.float32)],
    )(h_pad, agg2, w1, b1, w2, b2, scale, batch3, wc1, bc1, wc2, bc2)


def kernel(x, edge_index, batch, emb, W1, b1, W2, b2, eps, Wc1, bc1, Wc2, bc2):
    n = x.shape[0]
    d = emb.shape[1]
    e = edge_index.shape[1]
    n_layers = W1.shape[0]
    blk = 1024

    n_pad = _round_up(n + 1, NW * 80)
    e_pad = _round_up(e, NS * ECH * 2)

    idx = jnp.concatenate(
        [x[:, 0], jnp.zeros((n_pad - n,), jnp.int32)])
    ei_pad = jnp.concatenate(
        [edge_index, jnp.full((2, e_pad - e), n, jnp.int32)], axis=1)
    edges_packed = ei_pad.reshape(2, e_pad // ECH, ECH).transpose(1, 0, 2)
    batch3 = jnp.concatenate(
        [batch, jnp.full((n_pad - n,), G, jnp.int32)]).reshape(
            n_pad // blk, 1, blk)

    h = _emb_gather(emb, idx, n_pad, d)
    for l in range(n_layers):
        agg2 = _edge_agg(h, edges_packed, n_pad, d)
        scale = (1.0 + eps[l]).reshape(1, 1)
        b1l = b1[l].reshape(1, d)
        b2l = b2[l].reshape(1, d)
        if l < n_layers - 1:
            h = _mlp(h, agg2, W1[l], b1l, W2[l], b2l, scale, n_pad, d, blk)
        else:
            score = _mlp_pool_cls(
                h, agg2, W1[l], b1l, W2[l], b2l, scale, batch3,
                Wc1, bc1.reshape(1, d), Wc2, bc2.reshape(1, 1),
                n_pad, d, blk)
    return score.reshape(-1)


# speedup vs baseline: 1.8335x; 1.5173x over previous
import functools

import jax
import jax.numpy as jnp
from jax import lax
from jax.experimental import pallas as pl
from jax.experimental.pallas import tpu as pltpu
from jax.experimental.pallas import tpu_sc as plsc

NC = 2
NS = 16
NW = NC * NS
G = 64
ECH = 128


def _round_up(a, m):
    return (a + m - 1) // m * m


FAST = 0
EMB_CH = 80
EMB_NF = 6
EMB_NS = 2


def _emb_gather(emb, idx, n_pad, d):
    mesh = plsc.VectorSubcoreMesh(core_axis_name="c", subcore_axis_name="s",
                                  num_cores=NC, num_subcores=NS)

    @functools.partial(
        pl.kernel,
        out_type=jax.ShapeDtypeStruct((n_pad, d), jnp.float32),
        mesh=mesh,
        scratch_types=[
            pltpu.VMEM((EMB_CH,), jnp.int32),
            pltpu.VMEM((EMB_CH, d), jnp.float32),
        ],
    )
    def k(emb_hbm, idx_hbm, h_hbm, idxv, rows):
        cid = lax.axis_index("c")
        sid = lax.axis_index("s")

        def run(base, nch):
            for c in range(nch):
                off = base + c * EMB_CH
                pltpu.sync_copy(idx_hbm.at[pl.ds(off, EMB_CH)], idxv)
                pltpu.sync_copy(emb_hbm.at[idxv], rows)
                pltpu.sync_copy(rows, h_hbm.at[pl.ds(off, EMB_CH)])

        @pl.when(cid == FAST)
        def _():
            run(sid * (EMB_NF * EMB_CH), EMB_NF)

        @pl.when(cid != FAST)
        def _():
            run(NS * EMB_NF * EMB_CH + sid * (EMB_NS * EMB_CH), EMB_NS)

    return k(emb, idx)


def _edge_agg(h_pad, edges_packed, n_pad, d):
    nch_total = edges_packed.shape[0]
    nchs = 12
    nchf = nch_total // NS - nchs
    rps = n_pad // NS
    mesh = plsc.VectorSubcoreMesh(core_axis_name="c", subcore_axis_name="s",
                                  num_cores=NC, num_subcores=NS)

    qmax = 60

    @functools.partial(
        pl.kernel,
        out_type=jax.ShapeDtypeStruct((NC, n_pad, d), jnp.float32),
        mesh=mesh,
        scratch_types=[
            pltpu.VMEM((qmax, 2, ECH), jnp.int32),
            pltpu.VMEM((ECH, d), jnp.float32),
            pltpu.VMEM((ECH, d), jnp.float32),
            pltpu.VMEM_SHARED((n_pad, d), jnp.float32),
            pltpu.SemaphoreType.DMA,
            pltpu.SemaphoreType.DMA,
        ],
    )
    def k(h_hbm, ei_hbm, out_hbm, iall, ra, rb, aggsh, sa, sb):
        cid = lax.axis_index("c")
        sid = lax.axis_index("s")

        def pipe_q(qoff, qn):
            pltpu.sync_copy(ei_hbm.at[pl.ds(qoff, qn)],
                            iall.at[pl.ds(0, qn)])
            pltpu.async_copy(h_hbm.at[iall.at[0, 0]], ra, sa)

            @pl.loop(0, (qn - 2) // 2)
            def _(j):
                a = 2 * j
                pltpu.async_copy(h_hbm.at[iall.at[a + 1, 0]], rb, sb)
                pltpu.make_async_copy(h_hbm.at[iall.at[a, 0]], ra, sa).wait()
                pltpu.sync_copy(ra, aggsh.at[iall.at[a, 1]], add=True)
                pltpu.async_copy(h_hbm.at[iall.at[a + 2, 0]], ra, sa)
                pltpu.make_async_copy(h_hbm.at[iall.at[a + 1, 0]], rb, sb).wait()
                pltpu.sync_copy(rb, aggsh.at[iall.at[a + 1, 1]], add=True)

            pltpu.async_copy(h_hbm.at[iall.at[qn - 1, 0]], rb, sb)
            pltpu.make_async_copy(h_hbm.at[iall.at[qn - 2, 0]], ra, sa).wait()
            pltpu.sync_copy(ra, aggsh.at[iall.at[qn - 2, 1]], add=True)
            pltpu.make_async_copy(h_hbm.at[iall.at[qn - 1, 0]], rb, sb).wait()
            pltpu.sync_copy(rb, aggsh.at[iall.at[qn - 1, 1]], add=True)

        def pipe(base, nchw):
            done = 0
            while done < nchw:
                qn = min(qmax, nchw - done)
                pipe_q(base + done, qn)
                done += qn

        zv = jnp.zeros((16,), jnp.float32)

        @pl.loop(0, ECH)
        def _(i):
            for j in range(d // 16):
                ra[i, pl.ds(j * 16, 16)] = zv

        for r in range(rps // ECH):
            pltpu.sync_copy(ra, aggsh.at[pl.ds(sid * rps + r * ECH, ECH)])
        plsc.subcore_barrier()

        @pl.when(cid == FAST)
        def _():
            pipe(sid * nchf, nchf)

        @pl.when(cid != FAST)
        def _():
            pipe(NS * nchf + sid * nchs, nchs)

        plsc.subcore_barrier()
        pltpu.sync_copy(aggsh.at[pl.ds(sid * rps, rps)],
                        out_hbm.at[cid, pl.ds(sid * rps, rps)])

    return k(h_pad, edges_packed)


def _mlp_body(h_ref, a_ref, w1_ref, b1_ref, w2_ref, b2_ref, sc_ref, o_ref):
    s = sc_ref[0, 0]
    z = h_ref[...] * s + a_ref[0] + a_ref[1]
    z = jnp.maximum(
        jnp.dot(z, w1_ref[...], preferred_element_type=jnp.float32)
        + b1_ref[...], 0.0)
    z = jnp.maximum(
        jnp.dot(z, w2_ref[...], preferred_element_type=jnp.float32)
        + b2_ref[...], 0.0)
    o_ref[...] = z


def _mlp(h_pad, agg2, w1, b1, w2, b2, scale, n_pad, d, blk):
    nb = n_pad // blk
    return pl.pallas_call(
        _mlp_body,
        grid=(nb,),
        in_specs=[
            pl.BlockSpec((blk, d), lambda i: (i, 0)),
            pl.BlockSpec((NC, blk, d), lambda i: (0, i, 0)),
            pl.BlockSpec((d, d), lambda i: (0, 0)),
            pl.BlockSpec((1, d), lambda i: (0, 0)),
            pl.BlockSpec((d, d), lambda i: (0, 0)),
            pl.BlockSpec((1, d), lambda i: (0, 0)),
            pl.BlockSpec(memory_space=pltpu.SMEM),
        ],
        out_specs=pl.BlockSpec((blk, d), lambda i: (i, 0)),
        out_shape=jax.ShapeDtypeStruct((n_pad, d), jnp.float32),
    )(h_pad, agg2, w1, b1, w2, b2, scale)


def _mlp_pool_cls(h_pad, agg2, w1, b1, w2, b2, scale, batch3, wc1, bc1,
                  wc2, bc2, n_pad, d, blk):
    nb = n_pad // blk

    def body(h_ref, a_ref, w1_ref, b1_ref, w2_ref, b2_ref, sc_ref, bt_ref,
             wc1_ref, bc1_ref, wc2_ref, bc2_ref, s_ref, hg_ref):
        i = pl.program_id(0)
        s = sc_ref[0, 0]
        z = h_ref[...] * s + a_ref[0] + a_ref[1]
        z = jnp.maximum(
            jnp.dot(z, w1_ref[...], preferred_element_type=jnp.float32)
            + b1_ref[...], 0.0)
        z = jnp.maximum(
            jnp.dot(z, w2_ref[...], preferred_element_type=jnp.float32)
            + b2_ref[...], 0.0)
        bt = bt_ref[0, 0, :]
        oh = (bt[:, None] == lax.broadcasted_iota(jnp.int32, (blk, G), 1))
        oh = oh.astype(jnp.float32)
        contrib = lax.dot_general(
            oh, z, (((0,), (0,)), ((), ())),
            preferred_element_type=jnp.float32)

        @pl.when(i == 0)
        def _():
            hg_ref[...] = jnp.zeros_like(hg_ref)

        hg_ref[...] += contrib

        @pl.when(i == nb - 1)
        def _():
            hg = hg_ref[...]
            hid = jnp.maximum(
                jnp.dot(hg, wc1_ref[...], preferred_element_type=jnp.float32)
                + bc1_ref[...], 0.0)
            logit = jnp.dot(hid, wc2_ref[...],
                            preferred_element_type=jnp.float32) + bc2_ref[0, 0]
            s_ref[...] = jax.nn.sigmoid(logit)

    return pl.pallas_call(
        body,
        grid=(nb,),
        in_specs=[
            pl.BlockSpec((blk, d), lambda i: (i, 0)),
            pl.BlockSpec((NC, blk, d), lambda i: (0, i, 0)),
            pl.BlockSpec((d, d), lambda i: (0, 0)),
            pl.BlockSpec((1, d), lambda i: (0, 0)),
            pl.BlockSpec((d, d), lambda i: (0, 0)),
            pl.BlockSpec((1, d), lambda i: (0, 0)),
            pl.BlockSpec(memory_space=pltpu.SMEM),
            pl.BlockSpec((1, 1, blk), lambda i: (i, 0, 0)),
            pl.BlockSpec((d, d), lambda i: (0, 0)),
            pl.BlockSpec((1, d), lambda i: (0, 0)),
            pl.BlockSpec((d, 1), lambda i: (0, 0)),
            pl.BlockSpec(memory_space=pltpu.SMEM),
        ],
        out_specs=pl.BlockSpec((G, 1), lambda i: (0, 0)),
        out_shape=jax.ShapeDtypeStruct((G, 1), jnp.float32),
        scratch_shapes=[pltpu.VMEM((G, d), jnp.float32)],
    )(h_pad, agg2, w1, b1, w2, b2, scale, batch3, wc1, bc1, wc2, bc2)


def kernel(x, edge_index, batch, emb, W1, b1, W2, b2, eps, Wc1, bc1, Wc2, bc2):
    n = x.shape[0]
    d = emb.shape[1]
    e = edge_index.shape[1]
    n_layers = W1.shape[0]
    blk = 1024

    n_pad = _round_up(n + 1, NW * 80)
    e_pad = _round_up(e, NS * ECH * 2)

    idx = jnp.concatenate(
        [x[:, 0], jnp.zeros((n_pad - n,), jnp.int32)])
    ei_pad = jnp.concatenate(
        [edge_index, jnp.full((2, e_pad - e), n, jnp.int32)], axis=1)
    edges_packed = ei_pad.reshape(2, e_pad // ECH, ECH).transpose(1, 0, 2)
    batch3 = jnp.concatenate(
        [batch, jnp.full((n_pad - n,), G, jnp.int32)]).reshape(
            n_pad // blk, 1, blk)

    h = _emb_gather(emb, idx, n_pad, d)
    for l in range(n_layers):
        agg2 = _edge_agg(h, edges_packed, n_pad, d)
        scale = (1.0 + eps[l]).reshape(1, 1)
        b1l = b1[l].reshape(1, d)
        b2l = b2[l].reshape(1, d)
        if l < n_layers - 1:
            h = _mlp(h, agg2, W1[l], b1l, W2[l], b2l, scale, n_pad, d, blk)
        else:
            score = _mlp_pool_cls(
                h, agg2, W1[l], b1l, W2[l], b2l, scale, batch3,
                Wc1, bc1.reshape(1, d), Wc2, bc2.reshape(1, 1),
                n_pad, d, blk)
    return score.reshape(-1)


# speedup vs baseline: 1.8715x; 1.0207x over previous
import functools

import jax
import jax.numpy as jnp
from jax import lax
from jax.experimental import pallas as pl
from jax.experimental.pallas import tpu as pltpu
from jax.experimental.pallas import tpu_sc as plsc

NC = 2
NS = 16
NW = NC * NS
G = 64
ECH = 128


def _round_up(a, m):
    return (a + m - 1) // m * m


FAST = 0
EMB_CH = 80
EMB_NF = 6
EMB_NS = 2


def _emb_gather(emb, idx, n_pad, d):
    mesh = plsc.VectorSubcoreMesh(core_axis_name="c", subcore_axis_name="s",
                                  num_cores=NC, num_subcores=NS)

    @functools.partial(
        pl.kernel,
        out_type=jax.ShapeDtypeStruct((n_pad, d), jnp.float32),
        mesh=mesh,
        scratch_types=[
            pltpu.VMEM((EMB_CH,), jnp.int32),
            pltpu.VMEM((EMB_CH, d), jnp.float32),
        ],
    )
    def k(emb_hbm, idx_hbm, h_hbm, idxv, rows):
        cid = lax.axis_index("c")
        sid = lax.axis_index("s")

        def run(base, nch):
            for c in range(nch):
                off = base + c * EMB_CH
                pltpu.sync_copy(idx_hbm.at[pl.ds(off, EMB_CH)], idxv)
                pltpu.sync_copy(emb_hbm.at[idxv], rows)
                pltpu.sync_copy(rows, h_hbm.at[pl.ds(off, EMB_CH)])

        @pl.when(cid == FAST)
        def _():
            run(sid * (EMB_NF * EMB_CH), EMB_NF)

        @pl.when(cid != FAST)
        def _():
            run(NS * EMB_NF * EMB_CH + sid * (EMB_NS * EMB_CH), EMB_NS)

    return k(emb, idx)


def _edge_agg(h_pad, edges_packed, n_pad, d):
    nch_total = edges_packed.shape[0]
    nchs = 10
    nchf = nch_total // NS - nchs
    rps = n_pad // NS
    mesh = plsc.VectorSubcoreMesh(core_axis_name="c", subcore_axis_name="s",
                                  num_cores=NC, num_subcores=NS)

    qmax = 60

    @functools.partial(
        pl.kernel,
        out_type=jax.ShapeDtypeStruct((NC, n_pad, d), jnp.float32),
        mesh=mesh,
        scratch_types=[
            pltpu.VMEM((qmax, 2, ECH), jnp.int32),
            pltpu.VMEM((ECH, d), jnp.float32),
            pltpu.VMEM((ECH, d), jnp.float32),
            pltpu.VMEM_SHARED((n_pad, d), jnp.float32),
            pltpu.SemaphoreType.DMA,
            pltpu.SemaphoreType.DMA,
        ],
    )
    def k(h_hbm, ei_hbm, out_hbm, iall, ra, rb, aggsh, sa, sb):
        cid = lax.axis_index("c")
        sid = lax.axis_index("s")

        def pipe_q(qoff, qn):
            pltpu.sync_copy(ei_hbm.at[pl.ds(qoff, qn)],
                            iall.at[pl.ds(0, qn)])
            pltpu.async_copy(h_hbm.at[iall.at[0, 0]], ra, sa)

            @pl.loop(0, (qn - 2) // 2)
            def _(j):
                a = 2 * j
                pltpu.async_copy(h_hbm.at[iall.at[a + 1, 0]], rb, sb)
                pltpu.make_async_copy(h_hbm.at[iall.at[a, 0]], ra, sa).wait()
                pltpu.sync_copy(ra, aggsh.at[iall.at[a, 1]], add=True)
                pltpu.async_copy(h_hbm.at[iall.at[a + 2, 0]], ra, sa)
                pltpu.make_async_copy(h_hbm.at[iall.at[a + 1, 0]], rb, sb).wait()
                pltpu.sync_copy(rb, aggsh.at[iall.at[a + 1, 1]], add=True)

            pltpu.async_copy(h_hbm.at[iall.at[qn - 1, 0]], rb, sb)
            pltpu.make_async_copy(h_hbm.at[iall.at[qn - 2, 0]], ra, sa).wait()
            pltpu.sync_copy(ra, aggsh.at[iall.at[qn - 2, 1]], add=True)
            pltpu.make_async_copy(h_hbm.at[iall.at[qn - 1, 0]], rb, sb).wait()
            pltpu.sync_copy(rb, aggsh.at[iall.at[qn - 1, 1]], add=True)

        def pipe(base, nchw):
            done = 0
            while done < nchw:
                qn = min(qmax, nchw - done)
                pipe_q(base + done, qn)
                done += qn

        zv = jnp.zeros((16,), jnp.float32)

        @pl.loop(0, ECH)
        def _(i):
            for j in range(d // 16):
                ra[i, pl.ds(j * 16, 16)] = zv

        for r in range(rps // ECH):
            pltpu.sync_copy(ra, aggsh.at[pl.ds(sid * rps + r * ECH, ECH)])
        plsc.subcore_barrier()

        @pl.when(cid == FAST)
        def _():
            pipe(sid * nchf, nchf)

        @pl.when(cid != FAST)
        def _():
            pipe(NS * nchf + sid * nchs, nchs)

        plsc.subcore_barrier()
        pltpu.sync_copy(aggsh.at[pl.ds(sid * rps, rps)],
                        out_hbm.at[cid, pl.ds(sid * rps, rps)])

    return k(h_pad, edges_packed)


def _mlp_body(h_ref, a_ref, w1_ref, b1_ref, w2_ref, b2_ref, sc_ref, o_ref):
    s = sc_ref[0, 0]
    z = h_ref[...] * s + a_ref[0] + a_ref[1]
    z = jnp.maximum(
        jnp.dot(z, w1_ref[...], preferred_element_type=jnp.float32)
        + b1_ref[...], 0.0)
    z = jnp.maximum(
        jnp.dot(z, w2_ref[...], preferred_element_type=jnp.float32)
        + b2_ref[...], 0.0)
    o_ref[...] = z


def _mlp(h_pad, agg2, w1, b1, w2, b2, scale, n_pad, d, blk):
    nb = n_pad // blk
    return pl.pallas_call(
        _mlp_body,
        grid=(nb,),
        in_specs=[
            pl.BlockSpec((blk, d), lambda i: (i, 0)),
            pl.BlockSpec((NC, blk, d), lambda i: (0, i, 0)),
            pl.BlockSpec((d, d), lambda i: (0, 0)),
            pl.BlockSpec((1, d), lambda i: (0, 0)),
            pl.BlockSpec((d, d), lambda i: (0, 0)),
            pl.BlockSpec((1, d), lambda i: (0, 0)),
            pl.BlockSpec(memory_space=pltpu.SMEM),
        ],
        out_specs=pl.BlockSpec((blk, d), lambda i: (i, 0)),
        out_shape=jax.ShapeDtypeStruct((n_pad, d), jnp.float32),
    )(h_pad, agg2, w1, b1, w2, b2, scale)


def _mlp_pool_cls(h_pad, agg2, w1, b1, w2, b2, scale, batch3, wc1, bc1,
                  wc2, bc2, n_pad, d, blk):
    nb = n_pad // blk

    def body(h_ref, a_ref, w1_ref, b1_ref, w2_ref, b2_ref, sc_ref, bt_ref,
             wc1_ref, bc1_ref, wc2_ref, bc2_ref, s_ref, hg_ref):
        i = pl.program_id(0)
        s = sc_ref[0, 0]
        z = h_ref[...] * s + a_ref[0] + a_ref[1]
        z = jnp.maximum(
            jnp.dot(z, w1_ref[...], preferred_element_type=jnp.float32)
            + b1_ref[...], 0.0)
        z = jnp.maximum(
            jnp.dot(z, w2_ref[...], preferred_element_type=jnp.float32)
            + b2_ref[...], 0.0)
        bt = bt_ref[0, 0, :]
        oh = (bt[:, None] == lax.broadcasted_iota(jnp.int32, (blk, G), 1))
        oh = oh.astype(jnp.float32)
        contrib = lax.dot_general(
            oh, z, (((0,), (0,)), ((), ())),
            preferred_element_type=jnp.float32)

        @pl.when(i == 0)
        def _():
            hg_ref[...] = jnp.zeros_like(hg_ref)

        hg_ref[...] += contrib

        @pl.when(i == nb - 1)
        def _():
            hg = hg_ref[...]
            hid = jnp.maximum(
                jnp.dot(hg, wc1_ref[...], preferred_element_type=jnp.float32)
                + bc1_ref[...], 0.0)
            logit = jnp.dot(hid, wc2_ref[...],
                            preferred_element_type=jnp.float32) + bc2_ref[0, 0]
            s_ref[...] = jax.nn.sigmoid(logit)

    return pl.pallas_call(
        body,
        grid=(nb,),
        in_specs=[
            pl.BlockSpec((blk, d), lambda i: (i, 0)),
            pl.BlockSpec((NC, blk, d), lambda i: (0, i, 0)),
            pl.BlockSpec((d, d), lambda i: (0, 0)),
            pl.BlockSpec((1, d), lambda i: (0, 0)),
            pl.BlockSpec((d, d), lambda i: (0, 0)),
            pl.BlockSpec((1, d), lambda i: (0, 0)),
            pl.BlockSpec(memory_space=pltpu.SMEM),
            pl.BlockSpec((1, 1, blk), lambda i: (i, 0, 0)),
            pl.BlockSpec((d, d), lambda i: (0, 0)),
            pl.BlockSpec((1, d), lambda i: (0, 0)),
            pl.BlockSpec((d, 1), lambda i: (0, 0)),
            pl.BlockSpec(memory_space=pltpu.SMEM),
        ],
        out_specs=pl.BlockSpec((G, 1), lambda i: (0, 0)),
        out_shape=jax.ShapeDtypeStruct((G, 1), jnp.float32),
        scratch_shapes=[pltpu.VMEM((G, d), jnp.float32)],
    )(h_pad, agg2, w1, b1, w2, b2, scale, batch3, wc1, bc1, wc2, bc2)


def kernel(x, edge_index, batch, emb, W1, b1, W2, b2, eps, Wc1, bc1, Wc2, bc2):
    n = x.shape[0]
    d = emb.shape[1]
    e = edge_index.shape[1]
    n_layers = W1.shape[0]
    blk = 1024

    n_pad = _round_up(n + 1, NW * 80)
    e_pad = _round_up(e, NS * ECH * 2)

    idx = jnp.concatenate(
        [x[:, 0], jnp.zeros((n_pad - n,), jnp.int32)])
    ei_pad = jnp.concatenate(
        [edge_index, jnp.full((2, e_pad - e), n, jnp.int32)], axis=1)
    edges_packed = ei_pad.reshape(2, e_pad // ECH, ECH).transpose(1, 0, 2)
    batch3 = jnp.concatenate(
        [batch, jnp.full((n_pad - n,), G, jnp.int32)]).reshape(
            n_pad // blk, 1, blk)

    h = _emb_gather(emb, idx, n_pad, d)
    for l in range(n_layers):
        agg2 = _edge_agg(h, edges_packed, n_pad, d)
        scale = (1.0 + eps[l]).reshape(1, 1)
        b1l = b1[l].reshape(1, d)
        b2l = b2[l].reshape(1, d)
        if l < n_layers - 1:
            h = _mlp(h, agg2, W1[l], b1l, W2[l], b2l, scale, n_pad, d, blk)
        else:
            score = _mlp_pool_cls(
                h, agg2, W1[l], b1l, W2[l], b2l, scale, batch3,
                Wc1, bc1.reshape(1, d), Wc2, bc2.reshape(1, 1),
                n_pad, d, blk)
    return score.reshape(-1)
